# baseline (speedup 1.0000x reference)
import functools

import jax
import jax.numpy as jnp
from jax import lax
from jax.experimental import pallas as pl
from jax.experimental.pallas import tpu as pltpu
from jax.experimental.pallas import tpu_sc as plsc

NCORES = 2
NSUB = 16
NW = NCORES * NSUB
CH = 128
ZR = 104
NBUF = 6


def _make_s_kernel(n, e, w, gather):
    tot_chunks = e // CH
    base_chunks = tot_chunks // NW
    rem = tot_chunks - base_chunks * NW
    maxc = base_chunks + (1 if rem else 0)
    exp_rows = (n // NSUB) // 8 * 8
    tail_rows = n - exp_rows * NSUB
    assert exp_rows % ZR == 0 and tail_rows % 8 == 0 and tail_rows <= CH

    mesh = plsc.VectorSubcoreMesh(core_axis_name="c", subcore_axis_name="s")

    scratch = [
        pltpu.VMEM((maxc, CH), jnp.int32),
        pltpu.VMEM((maxc, CH), jnp.int32),
        pltpu.VMEM((NBUF, CH, w), jnp.float32),
        pltpu.VMEM_SHARED((n, w), jnp.float32),
        pltpu.SemaphoreType.DMA((NBUF,)),
        pltpu.SemaphoreType.DMA((NBUF,)),
    ]

    def body(*refs):
        if gather:
            (z_hbm, g_hbm, ei_hbm, out_hbm, colbig, rowbig, rows, acc,
             gsem, ssem) = refs
        else:
            (z_hbm, ei_hbm, out_hbm, colbig, rowbig, rows, acc,
             gsem, ssem) = refs
            g_hbm = None
        cid = lax.axis_index("c")
        sid = lax.axis_index("s")
        wid = sid * NCORES + cid

        row0 = sid * exp_rows
        pltpu.sync_copy(z_hbm.at[pl.ds(row0, exp_rows)],
                        acc.at[pl.ds(row0, exp_rows)])

        @pl.when(sid == NSUB - 1)
        def _():
            pltpu.sync_copy(z_hbm.at[pl.ds(exp_rows * NSUB, tail_rows)],
                            acc.at[pl.ds(exp_rows * NSUB, tail_rows)])

        start = base_chunks * wid + jnp.minimum(wid, rem)
        nch = base_chunks + (wid < rem).astype(jnp.int32)
        pltpu.sync_copy(ei_hbm.at[0, pl.ds(start, base_chunks)],
                        rowbig.at[pl.ds(0, base_chunks)])
        if gather:
            pltpu.sync_copy(ei_hbm.at[1, pl.ds(start, base_chunks)],
                            colbig.at[pl.ds(0, base_chunks)])
        if rem:
            @pl.when(wid < rem)
            def _():
                pltpu.sync_copy(ei_hbm.at[0, pl.ds(start + base_chunks, 1)],
                                rowbig.at[pl.ds(base_chunks, 1)])
                if gather:
                    pltpu.sync_copy(ei_hbm.at[1, pl.ds(start + base_chunks, 1)],
                                    colbig.at[pl.ds(base_chunks, 1)])

        plsc.subcore_barrier()

        if gather:
            for b in range(NBUF - 1):
                pltpu.async_copy(g_hbm.at[colbig.at[b]], rows.at[b],
                                 gsem.at[b])

            def do_chunk(j, carry):
                b = lax.rem(j, NBUF)
                pltpu.make_async_copy(g_hbm.at[colbig.at[j]], rows.at[b],
                                      gsem.at[b]).wait()
                pltpu.async_copy(rows.at[b], acc.at[rowbig.at[j]],
                                 ssem.at[b], add=True)
                jn = j + NBUF - 1
                bn = lax.rem(jn, NBUF)

                @pl.when(j > 0)
                def _():
                    pltpu.make_async_copy(
                        rows.at[bn], acc.at[rowbig.at[j - 1]],
                        ssem.at[bn]).wait()

                @pl.when(jn < nch)
                def _():
                    pltpu.async_copy(g_hbm.at[colbig.at[jn]], rows.at[bn],
                                     gsem.at[bn])
                return carry

            lax.fori_loop(0, nch, do_chunk, 0)
            bl = lax.rem(nch - 1, NBUF)
            pltpu.make_async_copy(rows.at[bl], acc.at[rowbig.at[nch - 1]],
                                  ssem.at[bl]).wait()
        else:
            one16 = jnp.ones((16,), jnp.float32)

            def orow(r, carry):
                for cc in range(w // 16):
                    rows[0, r, pl.ds(cc * 16, 16)] = one16
                return carry

            lax.fori_loop(0, CH, orow, 0)

            def do_chunk(j, carry):
                pltpu.sync_copy(rows.at[0], acc.at[rowbig.at[j]], add=True)
                return carry

            lax.fori_loop(0, nch, do_chunk, 0)

        plsc.subcore_barrier()
        pltpu.sync_copy(acc.at[pl.ds(row0, exp_rows)],
                        out_hbm.at[cid, pl.ds(row0, exp_rows)])

        @pl.when(sid == NSUB - 1)
        def _():
            pltpu.sync_copy(acc.at[pl.ds(exp_rows * NSUB, tail_rows)],
                            out_hbm.at[cid, pl.ds(exp_rows * NSUB, tail_rows)])

    return pl.kernel(
        body,
        out_type=jax.ShapeDtypeStruct((NCORES, n, w), jnp.float32),
        mesh=mesh,
        scratch_types=scratch,
        compiler_params=pltpu.CompilerParams(use_tc_tiling_on_sc=False),
    )


WC = 24


def _make_layer_kernel(n, e):
    tot_chunks = e // CH
    base_chunks = tot_chunks // NSUB
    rem = tot_chunks - base_chunks * NSUB
    maxc = base_chunks + (1 if rem else 0)
    exp_rows = (n // NSUB) // 8 * 8
    tail_rows = n - exp_rows * NSUB
    nzc = exp_rows // ZR

    mesh = plsc.VectorSubcoreMesh(core_axis_name="c", subcore_axis_name="s")

    plane = jax.ShapeDtypeStruct((NCORES * n, WC), jnp.float32)
    scratch = [
        pltpu.VMEM((maxc, CH), jnp.int32),
        pltpu.VMEM((maxc, CH), jnp.int32),
        pltpu.VMEM((NBUF, CH, WC), jnp.float32),
        pltpu.VMEM_SHARED((n, WC), jnp.float32),
        pltpu.VMEM((exp_rows + CH, WC), jnp.float32),
        pltpu.VMEM((ZR, WC), jnp.float32),
        pltpu.VMEM((ZR, WC), jnp.float32),
        pltpu.VMEM((ZR, WC), jnp.float32),
        pltpu.VMEM((ZR, WC), jnp.float32),
        pltpu.VMEM((ZR, WC), jnp.float32),
        pltpu.SemaphoreType.DMA((NBUF,)),
        pltpu.SemaphoreType.DMA((NBUF,)),
    ]

    def body(z_hbm, yd1, yd2, yd3, yd4, yd5, disb, ei_hbm,
             s5o, s3o, s1o, ga, gb,
             colbig, rowbig, rows, acc, dres,
             sbuf, ybuf, pbuf, gout, zbuf, gsem, ssem):
        cid = lax.axis_index("c")
        sid = lax.axis_index("s")
        row0 = sid * exp_rows
        is_tail = sid == NSUB - 1
        trow = exp_rows * NSUB
        pltpu.sync_copy(z_hbm.at[pl.ds(row0, ZR)], zbuf)
        pltpu.sync_copy(z_hbm.at[pl.ds(row0, exp_rows)],
                        acc.at[pl.ds(row0, exp_rows)])
        pltpu.sync_copy(disb.at[pl.ds(row0, exp_rows)],
                        dres.at[pl.ds(0, exp_rows)])

        @pl.when(is_tail)
        def _():
            pltpu.sync_copy(z_hbm.at[pl.ds(trow, tail_rows)],
                            acc.at[pl.ds(trow, tail_rows)])
            pltpu.sync_copy(disb.at[pl.ds(trow, tail_rows)],
                            dres.at[pl.ds(exp_rows, tail_rows)])

        start = base_chunks * sid + jnp.minimum(sid, rem)
        nch = base_chunks + (sid < rem).astype(jnp.int32)
        pltpu.sync_copy(ei_hbm.at[0, pl.ds(start, base_chunks)],
                        rowbig.at[pl.ds(0, base_chunks)])
        pltpu.sync_copy(ei_hbm.at[1, pl.ds(start, base_chunks)],
                        colbig.at[pl.ds(0, base_chunks)])
        if rem:
            @pl.when(sid < rem)
            def _():
                pltpu.sync_copy(ei_hbm.at[0, pl.ds(start + base_chunks, 1)],
                                rowbig.at[pl.ds(base_chunks, 1)])
                pltpu.sync_copy(ei_hbm.at[1, pl.ds(start + base_chunks, 1)],
                                colbig.at[pl.ds(base_chunks, 1)])

        offv = jnp.zeros((16,), jnp.int32) + cid * n

        def addrow(j, carry):
            for c in range(CH // 16):
                colbig[j, pl.ds(c * 16, 16)] = (
                    colbig[j, pl.ds(c * 16, 16)] + offv)
            return carry

        lax.fori_loop(0, maxc, addrow, 0)
        plsc.subcore_barrier()

        def s_round(src):
            for b in range(NBUF - 1):
                pltpu.async_copy(src.at[colbig.at[b]], rows.at[b],
                                 gsem.at[b])

            def do_chunk(j, carry):
                b = lax.rem(j, NBUF)
                pltpu.make_async_copy(src.at[colbig.at[j]], rows.at[b],
                                      gsem.at[b]).wait()
                pltpu.async_copy(rows.at[b], acc.at[rowbig.at[j]],
                                 ssem.at[b], add=True)
                jn = j + NBUF - 1
                bn = lax.rem(jn, NBUF)

                @pl.when(j > 0)
                def _():
                    pltpu.make_async_copy(
                        rows.at[bn], acc.at[rowbig.at[j - 1]],
                        ssem.at[bn]).wait()

                @pl.when(jn < nch)
                def _():
                    pltpu.async_copy(src.at[colbig.at[jn]], rows.at[bn],
                                     gsem.at[bn])
                return carry

            lax.fori_loop(0, nch, do_chunk, 0)
            bl = lax.rem(nch - 1, NBUF)
            pltpu.make_async_copy(rows.at[bl], acc.at[rowbig.at[nch - 1]],
                                  ssem.at[bl]).wait()

        def export(dst):
            pltpu.sync_copy(acc.at[pl.ds(row0, exp_rows)],
                            dst.at[pl.ds(cid * n + row0, exp_rows)])

            @pl.when(is_tail)
            def _():
                pltpu.sync_copy(acc.at[pl.ds(trow, tail_rows)],
                                dst.at[pl.ds(cid * n + trow, tail_rows)])

        def node_chunk(r0, nr, doff, ysrc, psrc, dst):
            pr = cid * n + r0
            pltpu.sync_copy(acc.at[pl.ds(r0, nr)], sbuf.at[pl.ds(0, nr)])
            pltpu.sync_copy(ysrc.at[pl.ds(pr, nr)], ybuf.at[pl.ds(0, nr)])
            if psrc is not None:
                pltpu.sync_copy(psrc.at[pl.ds(pr, nr)], pbuf.at[pl.ds(0, nr)])

            def vrow(r, carry):
                for o in (0, WC - 16):
                    val = (ybuf[r, pl.ds(o, 16)]
                           - dres[doff + r, pl.ds(o, 16)]
                           * sbuf[r, pl.ds(o, 16)])
                    if psrc is not None:
                        val = val - pbuf[r, pl.ds(o, 16)]
                    gout[r, pl.ds(o, 16)] = val
                return carry

            lax.fori_loop(0, nr, vrow, 0)
            pltpu.sync_copy(gout.at[pl.ds(0, nr)], dst.at[pl.ds(pr, nr)])
            pltpu.sync_copy(zbuf.at[pl.ds(0, nr)], acc.at[pl.ds(r0, nr)])

        def node_phase(ysrc, psrc, dst):
            for c in range(nzc):
                node_chunk(row0 + c * ZR, ZR, c * ZR, ysrc, psrc, dst)

            @pl.when(is_tail)
            def _():
                node_chunk(trow, tail_rows, exp_rows, ysrc, psrc, dst)

        s_round(yd5)
        plsc.subcore_barrier()
        export(s5o)
        node_phase(yd4, None, ga)
        plsc.subcore_barrier()

        s_round(ga)
        plsc.subcore_barrier()
        node_phase(yd3, yd5, gb)
        plsc.subcore_barrier()

        s_round(gb)
        plsc.subcore_barrier()
        export(s3o)
        node_phase(yd2, ga, ga)
        plsc.subcore_barrier()

        s_round(ga)
        plsc.subcore_barrier()
        node_phase(yd1, gb, gb)
        plsc.subcore_barrier()

        s_round(gb)
        plsc.subcore_barrier()
        export(s1o)

    return pl.kernel(
        body,
        out_type=(plane, plane, plane, plane, plane),
        mesh=mesh,
        scratch_types=scratch,
        compiler_params=pltpu.CompilerParams(use_tc_tiling_on_sc=False),
    )


def _make_mm_kernel(n, din, dout, blk=1000):
    def body(h_ref, w_ref, b_ref, d_ref, y_ref, yd_ref):
        y = jnp.dot(h_ref[...], w_ref[...],
                    preferred_element_type=jnp.float32) + b_ref[...]
        y_ref[...] = y
        yd_ref[...] = y * d_ref[...]

    return pl.pallas_call(
        body,
        grid=(n // blk,),
        in_specs=[pl.BlockSpec((blk, din), lambda i: (i, 0)),
                  pl.BlockSpec((din, dout), lambda i: (0, 0)),
                  pl.BlockSpec((1, dout), lambda i: (0, 0)),
                  pl.BlockSpec((blk, 1), lambda i: (i, 0))],
        out_specs=[pl.BlockSpec((blk, dout), lambda i: (i, 0)),
                   pl.BlockSpec((blk, dout), lambda i: (i, 0))],
        out_shape=[jax.ShapeDtypeStruct((n, dout), jnp.float32),
                   jax.ShapeDtypeStruct((n, dout), jnp.float32)],
    )


def _padw(Wt, dinp, doutp):
    K, din, dout = Wt.shape
    return jnp.zeros((K, dinp, doutp), jnp.float32).at[:, :din, :dout].set(Wt)


def _padv(b, doutp):
    return jnp.zeros((doutp,), jnp.float32).at[:b.shape[0]].set(b)


def kernel(x, edge_index, Wk0, bk0, Ws0, bs0, Wk1, bk1, Ws1, bs1,
           Wk2, bk2, Ws2, bs2, Wm, bm):
    n, d = x.shape
    e = edge_index.shape[1]
    nh = Wk0.shape[2]
    wp = 40
    wf = 16

    layerk = _make_layer_kernel(n, e)
    sF = _make_s_kernel(n, e, wf, gather=True)
    degk = _make_s_kernel(n, e, wf, gather=False)

    zf = jnp.zeros((n, wf), jnp.float32)
    ei3 = edge_index.reshape(2, e // CH, CH)

    degp = degk(zf, ei3)
    deg = degp[0, :, 0] + degp[1, :, 0]
    dis = jnp.where(deg > 0, lax.rsqrt(jnp.maximum(deg, 1e-12)), 0.0)
    disc = dis[:, None]
    tdisc2 = 2.0 * disc * disc

    def flat(a):
        bot = jnp.zeros((n, WC), jnp.float32).at[:, :wp - WC].set(a[:, WC:])
        return jnp.concatenate([a[:, :WC], bot], axis=0)

    def unflat(p):
        return jnp.concatenate([p[:n], p[n:, :wp - WC]], axis=1)

    disb = jnp.broadcast_to(tdisc2, (n, WC))
    zw = jnp.zeros((n, WC), jnp.float32)

    h = x
    for (Wk, bk, Ws, bs) in ((Wk0, bk0, Ws0, bs0), (Wk1, bk1, Ws1, bs1),
                             (Wk2, bk2, Ws2, bs2)):
        dinp = h.shape[1]
        Wkp = _padw(Wk, dinp, wp)
        Wcat = jnp.concatenate(
            [jnp.transpose(Wkp, (1, 0, 2)).reshape(dinp, 6 * wp),
             _padw(Ws, dinp, wp)[0]], axis=1)
        Bcat = (jnp.zeros((1, 7 * wp), jnp.float32)
                .at[0, :wp].set(_padv(bk, wp))
                .at[0, 6 * wp:].set(_padv(bs, wp)))
        Y, Yd = _make_mm_kernel(n, dinp, 7 * wp)(h, Wcat, Bcat, disc)
        y = [Y[:, k * wp:(k + 1) * wp] for k in range(6)]
        yd = [Yd[:, k * wp:(k + 1) * wp] for k in range(6)]
        s = Y[:, 6 * wp:]
        s5f, s3f, s1f, _, _ = layerk(zw, flat(yd[1]), flat(yd[2]),
                                     flat(yd[3]), flat(yd[4]), flat(yd[5]),
                                     disb, ei3)
        S5, S3, S1 = unflat(s5f), unflat(s3f), unflat(s1f)
        b4 = y[4] - 2.0 * disc * S5
        b2 = y[2] - 2.0 * disc * S3 - b4
        out6 = y[0] - disc * S1 - b2
        h = jax.nn.relu(out6) + s

    nc = Wm.shape[2]
    hc = jnp.concatenate([h, x], axis=1)
    dinf = hc.shape[1]
    wfin = (jnp.zeros((dinf, 2 * wf), jnp.float32)
            .at[:nh, :nc].set(Wm[1][:nh])
            .at[wp:, :nc].set(Wm[1][nh:])
            .at[:nh, wf:wf + nc].set(Wm[0][:nh])
            .at[wp:, wf:wf + nc].set(Wm[0][nh:]))
    bfin = (jnp.zeros((1, 2 * wf), jnp.float32)
            .at[0, wf:wf + nc].set(bm))
    Yf, Ydf = _make_mm_kernel(n, dinf, 2 * wf)(hc, wfin, bfin, disc)
    p = sF(zf, Ydf[:, :wf], ei3)
    sv = (p[0] + p[1])[:, :nc]
    return Yf[:, wf:wf + nc] - disc * sv


# speedup vs baseline: 1.2596x; 1.2596x over previous
import functools

import jax
import jax.numpy as jnp
from jax import lax
from jax.experimental import pallas as pl
from jax.experimental.pallas import tpu as pltpu
from jax.experimental.pallas import tpu_sc as plsc

NCORES = 2
NSUB = 16
NW = NCORES * NSUB
CH = 128
ZR = 104
NBUF = 10


def _make_s_kernel(n, e, w, gather):
    tot_chunks = e // CH
    base_chunks = tot_chunks // NW
    rem = tot_chunks - base_chunks * NW
    maxc = base_chunks + (1 if rem else 0)
    exp_rows = (n // NSUB) // 8 * 8
    tail_rows = n - exp_rows * NSUB
    assert exp_rows % ZR == 0 and tail_rows % 8 == 0 and tail_rows <= CH

    mesh = plsc.VectorSubcoreMesh(core_axis_name="c", subcore_axis_name="s")

    scratch = [
        pltpu.VMEM((maxc, CH), jnp.int32),
        pltpu.VMEM((maxc, CH), jnp.int32),
        pltpu.VMEM((NBUF, CH, w), jnp.float32),
        pltpu.VMEM_SHARED((n, w), jnp.float32),
        pltpu.SemaphoreType.DMA((NBUF,)),
        pltpu.SemaphoreType.DMA((NBUF,)),
    ]

    def body(*refs):
        if gather:
            (z_hbm, g_hbm, ei_hbm, out_hbm, colbig, rowbig, rows, acc,
             gsem, ssem) = refs
        else:
            (z_hbm, ei_hbm, out_hbm, colbig, rowbig, rows, acc,
             gsem, ssem) = refs
            g_hbm = None
        cid = lax.axis_index("c")
        sid = lax.axis_index("s")
        wid = sid * NCORES + cid

        row0 = sid * exp_rows
        pltpu.sync_copy(z_hbm.at[pl.ds(row0, exp_rows)],
                        acc.at[pl.ds(row0, exp_rows)])

        @pl.when(sid == NSUB - 1)
        def _():
            pltpu.sync_copy(z_hbm.at[pl.ds(exp_rows * NSUB, tail_rows)],
                            acc.at[pl.ds(exp_rows * NSUB, tail_rows)])

        start = base_chunks * wid + jnp.minimum(wid, rem)
        nch = base_chunks + (wid < rem).astype(jnp.int32)
        pltpu.sync_copy(ei_hbm.at[0, pl.ds(start, base_chunks)],
                        rowbig.at[pl.ds(0, base_chunks)])
        if gather:
            pltpu.sync_copy(ei_hbm.at[1, pl.ds(start, base_chunks)],
                            colbig.at[pl.ds(0, base_chunks)])
        if rem:
            @pl.when(wid < rem)
            def _():
                pltpu.sync_copy(ei_hbm.at[0, pl.ds(start + base_chunks, 1)],
                                rowbig.at[pl.ds(base_chunks, 1)])
                if gather:
                    pltpu.sync_copy(ei_hbm.at[1, pl.ds(start + base_chunks, 1)],
                                    colbig.at[pl.ds(base_chunks, 1)])

        plsc.subcore_barrier()

        if gather:
            for b in range(NBUF - 1):
                pltpu.async_copy(g_hbm.at[colbig.at[b]], rows.at[b],
                                 gsem.at[b])

            def do_chunk(j, carry):
                b = lax.rem(j, NBUF)
                pltpu.make_async_copy(g_hbm.at[colbig.at[j]], rows.at[b],
                                      gsem.at[b]).wait()
                pltpu.async_copy(rows.at[b], acc.at[rowbig.at[j]],
                                 ssem.at[b], add=True)
                jn = j + NBUF - 1
                bn = lax.rem(jn, NBUF)

                @pl.when(j > 0)
                def _():
                    pltpu.make_async_copy(
                        rows.at[bn], acc.at[rowbig.at[j - 1]],
                        ssem.at[bn]).wait()

                @pl.when(jn < nch)
                def _():
                    pltpu.async_copy(g_hbm.at[colbig.at[jn]], rows.at[bn],
                                     gsem.at[bn])
                return carry

            lax.fori_loop(0, nch, do_chunk, 0)
            bl = lax.rem(nch - 1, NBUF)
            pltpu.make_async_copy(rows.at[bl], acc.at[rowbig.at[nch - 1]],
                                  ssem.at[bl]).wait()
        else:
            one16 = jnp.ones((16,), jnp.float32)

            def orow(r, carry):
                for cc in range(w // 16):
                    rows[0, r, pl.ds(cc * 16, 16)] = one16
                return carry

            lax.fori_loop(0, CH, orow, 0)

            def do_chunk(j, carry):
                pltpu.sync_copy(rows.at[0], acc.at[rowbig.at[j]], add=True)
                return carry

            lax.fori_loop(0, nch, do_chunk, 0)

        plsc.subcore_barrier()
        pltpu.sync_copy(acc.at[pl.ds(row0, exp_rows)],
                        out_hbm.at[cid, pl.ds(row0, exp_rows)])

        @pl.when(sid == NSUB - 1)
        def _():
            pltpu.sync_copy(acc.at[pl.ds(exp_rows * NSUB, tail_rows)],
                            out_hbm.at[cid, pl.ds(exp_rows * NSUB, tail_rows)])

    return pl.kernel(
        body,
        out_type=jax.ShapeDtypeStruct((NCORES, n, w), jnp.float32),
        mesh=mesh,
        scratch_types=scratch,
        compiler_params=pltpu.CompilerParams(use_tc_tiling_on_sc=False),
    )


def _make_mm_kernel(n, din, dout, blk=1000):
    def body(h_ref, w_ref, b_ref, d_ref, y_ref, yd_ref):
        y = jnp.dot(h_ref[...], w_ref[...],
                    preferred_element_type=jnp.float32) + b_ref[...]
        y_ref[...] = y
        yd_ref[...] = y * d_ref[...]

    return pl.pallas_call(
        body,
        grid=(n // blk,),
        in_specs=[pl.BlockSpec((blk, din), lambda i: (i, 0)),
                  pl.BlockSpec((din, dout), lambda i: (0, 0)),
                  pl.BlockSpec((1, dout), lambda i: (0, 0)),
                  pl.BlockSpec((blk, 1), lambda i: (i, 0))],
        out_specs=[pl.BlockSpec((blk, dout), lambda i: (i, 0)),
                   pl.BlockSpec((blk, dout), lambda i: (i, 0))],
        out_shape=[jax.ShapeDtypeStruct((n, dout), jnp.float32),
                   jax.ShapeDtypeStruct((n, dout), jnp.float32)],
    )


def _padw(Wt, dinp, doutp):
    K, din, dout = Wt.shape
    return jnp.zeros((K, dinp, doutp), jnp.float32).at[:, :din, :dout].set(Wt)


def _padv(b, doutp):
    return jnp.zeros((doutp,), jnp.float32).at[:b.shape[0]].set(b)


def kernel(x, edge_index, Wk0, bk0, Ws0, bs0, Wk1, bk1, Ws1, bs1,
           Wk2, bk2, Ws2, bs2, Wm, bm):
    n, d = x.shape
    e = edge_index.shape[1]
    nh = Wk0.shape[2]
    wp = 40
    wf = 16

    sP = _make_s_kernel(n, e, wp, gather=True)
    sF = _make_s_kernel(n, e, wf, gather=True)
    degk = _make_s_kernel(n, e, wf, gather=False)

    zp = jnp.zeros((n, wp), jnp.float32)
    zf = jnp.zeros((n, wf), jnp.float32)
    ei3 = edge_index.reshape(2, e // CH, CH)

    degp = degk(zf, ei3)
    deg = degp[0, :, 0] + degp[1, :, 0]
    dis = jnp.where(deg > 0, lax.rsqrt(jnp.maximum(deg, 1e-12)), 0.0)
    disc = dis[:, None]
    tdisc2 = 2.0 * disc * disc

    def S(g):
        p = sP(zp, g, ei3)
        return p[0] + p[1]

    h = x
    for (Wk, bk, Ws, bs) in ((Wk0, bk0, Ws0, bs0), (Wk1, bk1, Ws1, bs1),
                             (Wk2, bk2, Ws2, bs2)):
        dinp = h.shape[1]
        Wkp = _padw(Wk, dinp, wp)
        Wcat = jnp.concatenate(
            [jnp.transpose(Wkp, (1, 0, 2)).reshape(dinp, 6 * wp),
             _padw(Ws, dinp, wp)[0]], axis=1)
        Bcat = (jnp.zeros((1, 7 * wp), jnp.float32)
                .at[0, :wp].set(_padv(bk, wp))
                .at[0, 6 * wp:].set(_padv(bs, wp)))
        Y, Yd = _make_mm_kernel(n, dinp, 7 * wp)(h, Wcat, Bcat, disc)
        y = [Y[:, k * wp:(k + 1) * wp] for k in range(6)]
        yd = [Yd[:, k * wp:(k + 1) * wp] for k in range(6)]
        s = Y[:, 6 * wp:]
        g5 = yd[5]
        S5 = S(g5)
        g4 = yd[4] - tdisc2 * S5
        S4 = S(g4)
        g3 = yd[3] - tdisc2 * S4 - g5
        S3 = S(g3)
        g2 = yd[2] - tdisc2 * S3 - g4
        S2 = S(g2)
        g1 = yd[1] - tdisc2 * S2 - g3
        S1 = S(g1)
        b4 = y[4] - 2.0 * disc * S5
        b2 = y[2] - 2.0 * disc * S3 - b4
        out6 = y[0] - disc * S1 - b2
        h = jax.nn.relu(out6) + s

    nc = Wm.shape[2]
    hc = jnp.concatenate([h, x], axis=1)
    dinf = hc.shape[1]
    wfin = (jnp.zeros((dinf, 2 * wf), jnp.float32)
            .at[:nh, :nc].set(Wm[1][:nh])
            .at[wp:, :nc].set(Wm[1][nh:])
            .at[:nh, wf:wf + nc].set(Wm[0][:nh])
            .at[wp:, wf:wf + nc].set(Wm[0][nh:]))
    bfin = (jnp.zeros((1, 2 * wf), jnp.float32)
            .at[0, wf:wf + nc].set(bm))
    Yf, Ydf = _make_mm_kernel(n, dinf, 2 * wf)(hc, wfin, bfin, disc)
    p = sF(zf, Ydf[:, :wf], ei3)
    sv = (p[0] + p[1])[:, :nc]
    return Yf[:, wf:wf + nc] - disc * sv


# speedup vs baseline: 1.2605x; 1.0008x over previous
import functools

import jax
import jax.numpy as jnp
from jax import lax
from jax.experimental import pallas as pl
from jax.experimental.pallas import tpu as pltpu
from jax.experimental.pallas import tpu_sc as plsc

NCORES = 2
NSUB = 16
NW = NCORES * NSUB
CH = 128
ZR = 104
NBUF = 6


def _make_s_kernel(n, e, w, gather):
    tot_chunks = e // CH
    base_chunks = tot_chunks // NW
    rem = tot_chunks - base_chunks * NW
    maxc = base_chunks + (1 if rem else 0)
    exp_rows = (n // NSUB) // 8 * 8
    tail_rows = n - exp_rows * NSUB
    assert exp_rows % ZR == 0 and tail_rows % 8 == 0 and tail_rows <= CH

    mesh = plsc.VectorSubcoreMesh(core_axis_name="c", subcore_axis_name="s")

    scratch = [
        pltpu.VMEM((maxc, CH), jnp.int32),
        pltpu.VMEM((maxc, CH), jnp.int32),
        pltpu.VMEM((NBUF, CH, w), jnp.float32),
        pltpu.VMEM_SHARED((n, w), jnp.float32),
        pltpu.SemaphoreType.DMA((NBUF,)),
        pltpu.SemaphoreType.DMA((NBUF,)),
    ]

    def body(*refs):
        if gather:
            (z_hbm, g_hbm, ei_hbm, out_hbm, colbig, rowbig, rows, acc,
             gsem, ssem) = refs
        else:
            (z_hbm, ei_hbm, out_hbm, colbig, rowbig, rows, acc,
             gsem, ssem) = refs
            g_hbm = None
        cid = lax.axis_index("c")
        sid = lax.axis_index("s")
        wid = sid * NCORES + cid

        row0 = sid * exp_rows
        pltpu.sync_copy(z_hbm.at[pl.ds(row0, exp_rows)],
                        acc.at[pl.ds(row0, exp_rows)])

        @pl.when(sid == NSUB - 1)
        def _():
            pltpu.sync_copy(z_hbm.at[pl.ds(exp_rows * NSUB, tail_rows)],
                            acc.at[pl.ds(exp_rows * NSUB, tail_rows)])

        start = base_chunks * wid + jnp.minimum(wid, rem)
        nch = base_chunks + (wid < rem).astype(jnp.int32)
        pltpu.sync_copy(ei_hbm.at[0, pl.ds(start, base_chunks)],
                        rowbig.at[pl.ds(0, base_chunks)])
        if gather:
            pltpu.sync_copy(ei_hbm.at[1, pl.ds(start, base_chunks)],
                            colbig.at[pl.ds(0, base_chunks)])
        if rem:
            @pl.when(wid < rem)
            def _():
                pltpu.sync_copy(ei_hbm.at[0, pl.ds(start + base_chunks, 1)],
                                rowbig.at[pl.ds(base_chunks, 1)])
                if gather:
                    pltpu.sync_copy(ei_hbm.at[1, pl.ds(start + base_chunks, 1)],
                                    colbig.at[pl.ds(base_chunks, 1)])

        plsc.subcore_barrier()

        if gather:
            for b in range(NBUF - 1):
                pltpu.async_copy(g_hbm.at[colbig.at[b]], rows.at[b],
                                 gsem.at[b])

            def do_chunk(j, carry):
                b = lax.rem(j, NBUF)
                pltpu.make_async_copy(g_hbm.at[colbig.at[j]], rows.at[b],
                                      gsem.at[b]).wait()
                pltpu.async_copy(rows.at[b], acc.at[rowbig.at[j]],
                                 ssem.at[b], add=True)
                jn = j + NBUF - 1
                bn = lax.rem(jn, NBUF)

                @pl.when(j > 0)
                def _():
                    pltpu.make_async_copy(
                        rows.at[bn], acc.at[rowbig.at[j - 1]],
                        ssem.at[bn]).wait()

                @pl.when(jn < nch)
                def _():
                    pltpu.async_copy(g_hbm.at[colbig.at[jn]], rows.at[bn],
                                     gsem.at[bn])
                return carry

            lax.fori_loop(0, nch, do_chunk, 0)
            bl = lax.rem(nch - 1, NBUF)
            pltpu.make_async_copy(rows.at[bl], acc.at[rowbig.at[nch - 1]],
                                  ssem.at[bl]).wait()
        else:
            one16 = jnp.ones((16,), jnp.float32)

            def orow(r, carry):
                for cc in range(w // 16):
                    rows[0, r, pl.ds(cc * 16, 16)] = one16
                return carry

            lax.fori_loop(0, CH, orow, 0)

            def do_chunk(j, carry):
                pltpu.sync_copy(rows.at[0], acc.at[rowbig.at[j]], add=True)
                return carry

            lax.fori_loop(0, nch, do_chunk, 0)

        plsc.subcore_barrier()
        pltpu.sync_copy(acc.at[pl.ds(row0, exp_rows)],
                        out_hbm.at[cid, pl.ds(row0, exp_rows)])

        @pl.when(sid == NSUB - 1)
        def _():
            pltpu.sync_copy(acc.at[pl.ds(exp_rows * NSUB, tail_rows)],
                            out_hbm.at[cid, pl.ds(exp_rows * NSUB, tail_rows)])

    return pl.kernel(
        body,
        out_type=jax.ShapeDtypeStruct((NCORES, n, w), jnp.float32),
        mesh=mesh,
        scratch_types=scratch,
        compiler_params=pltpu.CompilerParams(use_tc_tiling_on_sc=False),
    )


def _make_mm_kernel(n, din, dout, blk=1000):
    def body(h_ref, w_ref, b_ref, d_ref, y_ref, yd_ref):
        y = jnp.dot(h_ref[...], w_ref[...],
                    preferred_element_type=jnp.float32) + b_ref[...]
        y_ref[...] = y
        yd_ref[...] = y * d_ref[...]

    return pl.pallas_call(
        body,
        grid=(n // blk,),
        in_specs=[pl.BlockSpec((blk, din), lambda i: (i, 0)),
                  pl.BlockSpec((din, dout), lambda i: (0, 0)),
                  pl.BlockSpec((1, dout), lambda i: (0, 0)),
                  pl.BlockSpec((blk, 1), lambda i: (i, 0))],
        out_specs=[pl.BlockSpec((blk, dout), lambda i: (i, 0)),
                   pl.BlockSpec((blk, dout), lambda i: (i, 0))],
        out_shape=[jax.ShapeDtypeStruct((n, dout), jnp.float32),
                   jax.ShapeDtypeStruct((n, dout), jnp.float32)],
    )


def _padw(Wt, dinp, doutp):
    K, din, dout = Wt.shape
    return jnp.zeros((K, dinp, doutp), jnp.float32).at[:, :din, :dout].set(Wt)


def _padv(b, doutp):
    return jnp.zeros((doutp,), jnp.float32).at[:b.shape[0]].set(b)


def kernel(x, edge_index, Wk0, bk0, Ws0, bs0, Wk1, bk1, Ws1, bs1,
           Wk2, bk2, Ws2, bs2, Wm, bm):
    n, d = x.shape
    e = edge_index.shape[1]
    nh = Wk0.shape[2]
    wp = 40
    wf = 16

    sP = _make_s_kernel(n, e, wp, gather=True)
    sF = _make_s_kernel(n, e, wf, gather=True)
    degk = _make_s_kernel(n, e, wf, gather=False)

    zp = jnp.zeros((n, wp), jnp.float32)
    zf = jnp.zeros((n, wf), jnp.float32)
    ei3 = edge_index.reshape(2, e // CH, CH)

    degp = degk(zf, ei3)
    deg = degp[0, :, 0] + degp[1, :, 0]
    dis = jnp.where(deg > 0, lax.rsqrt(jnp.maximum(deg, 1e-12)), 0.0)
    disc = dis[:, None]
    tdisc2 = 2.0 * disc * disc

    def S(g):
        p = sP(zp, g, ei3)
        return p[0] + p[1]

    h = x
    for (Wk, bk, Ws, bs) in ((Wk0, bk0, Ws0, bs0), (Wk1, bk1, Ws1, bs1),
                             (Wk2, bk2, Ws2, bs2)):
        dinp = h.shape[1]
        Wkp = _padw(Wk, dinp, wp)
        Wcat = jnp.concatenate(
            [jnp.transpose(Wkp, (1, 0, 2)).reshape(dinp, 6 * wp),
             _padw(Ws, dinp, wp)[0]], axis=1)
        Bcat = (jnp.zeros((1, 7 * wp), jnp.float32)
                .at[0, :wp].set(_padv(bk, wp))
                .at[0, 6 * wp:].set(_padv(bs, wp)))
        Y, Yd = _make_mm_kernel(n, dinp, 7 * wp)(h, Wcat, Bcat, disc)
        y = [Y[:, k * wp:(k + 1) * wp] for k in range(6)]
        yd = [Yd[:, k * wp:(k + 1) * wp] for k in range(6)]
        s = Y[:, 6 * wp:]
        g5 = yd[5]
        S5 = S(g5)
        g4 = yd[4] - tdisc2 * S5
        S4 = S(g4)
        g3 = yd[3] - tdisc2 * S4 - g5
        S3 = S(g3)
        g2 = yd[2] - tdisc2 * S3 - g4
        S2 = S(g2)
        g1 = yd[1] - tdisc2 * S2 - g3
        S1 = S(g1)
        b4 = y[4] - 2.0 * disc * S5
        b2 = y[2] - 2.0 * disc * S3 - b4
        out6 = y[0] - disc * S1 - b2
        h = jax.nn.relu(out6) + s

    nc = Wm.shape[2]
    hc = jnp.concatenate([h, x], axis=1)
    dinf = hc.shape[1]
    wfin = (jnp.zeros((dinf, 2 * wf), jnp.float32)
            .at[:nh, :nc].set(Wm[1][:nh])
            .at[wp:, :nc].set(Wm[1][nh:])
            .at[:nh, wf:wf + nc].set(Wm[0][:nh])
            .at[wp:, wf:wf + nc].set(Wm[0][nh:]))
    bfin = (jnp.zeros((1, 2 * wf), jnp.float32)
            .at[0, wf:wf + nc].set(bm))
    Yf, Ydf = _make_mm_kernel(n, dinf, 2 * wf)(hc, wfin, bfin, disc)
    p = sF(zf, Ydf[:, :wf], ei3)
    sv = (p[0] + p[1])[:, :nc]
    return Yf[:, wf:wf + nc] - disc * sv
